# baseline (speedup 1.0000x reference)
import functools

import jax
import jax.numpy as jnp
from jax import lax
from jax.experimental import pallas as pl
from jax.experimental.pallas import tpu as pltpu
from jax.experimental.pallas import tpu_sc as plsc

H, W, D = 256, 256, 256
NC, NS, L = 2, 16, 16
NW = NC * NS
RPW = H // NW
HALF = 128
KD = D // L

_mesh = plsc.VectorSubcoreMesh(core_axis_name="c", subcore_axis_name="s")


@functools.partial(
    pl.kernel,
    out_type=jax.ShapeDtypeStruct((H * W, D), jnp.float32),
    mesh=_mesh,
    scratch_types=[
        pltpu.VMEM((NW, RPW), jnp.int32),
        pltpu.VMEM((2, HALF), jnp.int32),
        pltpu.VMEM((RPW, D), jnp.float32),
        pltpu.VMEM((W, D), jnp.float32),
        pltpu.VMEM((HALF, D), jnp.float32),
        pltpu.SemaphoreType.DMA,
    ],
)
def _pos_embed_sc(rows_hbm, cols_hbm, ht_hbm, wt_hbm, out_hbm,
                  ridx, cidx, h_buf, wt_buf, o_buf, sem):
    wid = lax.axis_index("s") * NC + lax.axis_index("c")
    pltpu.sync_copy(rows_hbm, ridx)
    pltpu.sync_copy(cols_hbm, cidx)
    pltpu.async_copy(ht_hbm.at[ridx.at[wid]], h_buf, sem).wait()
    pltpu.async_copy(wt_hbm.at[cidx.at[0]], wt_buf.at[pl.ds(0, HALF), :], sem).wait()
    pltpu.async_copy(wt_hbm.at[cidx.at[1]], wt_buf.at[pl.ds(HALF, HALF), :], sem).wait()

    for i in range(RPW):
        hv = [h_buf[i, pl.ds(L * k, L)] for k in range(KD)]
        for half in range(2):

            def body(j, carry, _half=half, _hv=hv):
                for k in range(KD):
                    o_buf[j, pl.ds(L * k, L)] = (
                        wt_buf[j + _half * HALF, pl.ds(L * k, L)] + _hv[k])
                return carry

            lax.fori_loop(0, HALF, body, 0)
            row0 = (wid * RPW + i) * W + half * HALF
            pltpu.sync_copy(o_buf, out_hbm.at[pl.ds(row0, HALF), :])


def kernel(height_table, width_table, shape):
    h = height_table.shape[0]
    w = width_table.shape[0]
    rows = jnp.minimum(jnp.arange(h, dtype=jnp.int32), shape[0] - 1)
    cols = jnp.minimum(jnp.arange(w, dtype=jnp.int32), shape[1] - 1)
    return _pos_embed_sc(
        rows.astype(jnp.int32).reshape(NW, RPW),
        cols.astype(jnp.int32).reshape(2, HALF),
        height_table, width_table)


# speedup vs baseline: 1.2011x; 1.2011x over previous
import functools

import jax
import jax.numpy as jnp
from jax import lax
from jax.experimental import pallas as pl
from jax.experimental.pallas import tpu as pltpu
from jax.experimental.pallas import tpu_sc as plsc

H, W, D = 256, 256, 256
NC, NS, L = 2, 16, 16
NW = NC * NS
RPW = H // NW
HALF = 128
QTR = 64
KD = D // L

_mesh = plsc.VectorSubcoreMesh(core_axis_name="c", subcore_axis_name="s")


@functools.partial(
    pl.kernel,
    out_type=jax.ShapeDtypeStruct((H * W, D), jnp.float32),
    mesh=_mesh,
    scratch_types=[
        pltpu.VMEM((NW, RPW), jnp.int32),
        pltpu.VMEM((2, HALF), jnp.int32),
        pltpu.VMEM((RPW, D), jnp.float32),
        pltpu.VMEM((W, D), jnp.float32),
        pltpu.VMEM((2, QTR, D), jnp.float32),
        pltpu.SemaphoreType.DMA,
        pltpu.SemaphoreType.DMA,
        pltpu.SemaphoreType.DMA,
        pltpu.SemaphoreType.DMA,
    ],
)
def _pos_embed_sc(rows_hbm, cols_hbm, ht_hbm, wt_hbm, out_hbm,
                  ridx, cidx, h_buf, wt_buf, o_buf, sem_g, sem_wg, sem0, sem1):
    wid = lax.axis_index("s") * NC + lax.axis_index("c")
    pltpu.sync_copy(rows_hbm, ridx)
    pltpu.sync_copy(cols_hbm, cidx)
    hrows = pltpu.async_copy(ht_hbm.at[ridx.at[wid]], h_buf, sem_g)
    wga = pltpu.async_copy(wt_hbm.at[cidx.at[0]], wt_buf.at[pl.ds(0, HALF), :], sem_g)
    wgb = pltpu.async_copy(wt_hbm.at[cidx.at[1]], wt_buf.at[pl.ds(HALF, HALF), :], sem_wg)
    hrows.wait()
    wga.wait()

    out_sems = (sem0, sem1)
    pending = [None, None]
    for c, (q, i) in enumerate([(q, i) for q in range(W // QTR) for i in range(RPW)]):
        if q == 2 and i == 0:
            wgb.wait()
        b = c % 2
        if pending[b] is not None:
            pending[b].wait()
        hv = [h_buf[i, pl.ds(L * k, L)] for k in range(KD)]

        def body(j, carry, _q=q, _hv=hv, _b=b):
            for k in range(KD):
                o_buf[_b, j, pl.ds(L * k, L)] = (
                    wt_buf[j + _q * QTR, pl.ds(L * k, L)] + _hv[k])
            return carry

        lax.fori_loop(0, QTR, body, 0)
        row0 = (wid * RPW + i) * W + q * QTR
        pending[b] = pltpu.async_copy(
            o_buf.at[b], out_hbm.at[pl.ds(row0, QTR), :], out_sems[b])
    pending[0].wait()
    pending[1].wait()


def kernel(height_table, width_table, shape):
    h = height_table.shape[0]
    w = width_table.shape[0]
    rows = jnp.minimum(jnp.arange(h, dtype=jnp.int32), shape[0] - 1)
    cols = jnp.minimum(jnp.arange(w, dtype=jnp.int32), shape[1] - 1)
    return _pos_embed_sc(
        rows.astype(jnp.int32).reshape(NW, RPW),
        cols.astype(jnp.int32).reshape(2, HALF),
        height_table, width_table)


# speedup vs baseline: 1.3637x; 1.1353x over previous
import functools

import jax
import jax.numpy as jnp
from jax import lax
from jax.experimental import pallas as pl
from jax.experimental.pallas import tpu as pltpu
from jax.experimental.pallas import tpu_sc as plsc

H, W, D = 256, 256, 256
NC, NS, L = 2, 16, 16
NW = NC * NS
RPW = H // NW
BH = 8

_mesh = plsc.VectorSubcoreMesh(core_axis_name="c", subcore_axis_name="s")


@functools.partial(
    pl.kernel,
    out_type=(jax.ShapeDtypeStruct((H, D), jnp.float32),
              jax.ShapeDtypeStruct((W, D), jnp.float32)),
    mesh=_mesh,
    scratch_types=[
        pltpu.VMEM((NW, RPW), jnp.int32),
        pltpu.VMEM((NW, RPW), jnp.int32),
        pltpu.VMEM((RPW, D), jnp.float32),
        pltpu.VMEM((RPW, D), jnp.float32),
        pltpu.SemaphoreType.DMA,
        pltpu.SemaphoreType.DMA,
    ],
)
def _lookup_sc(rows_hbm, cols_hbm, ht_hbm, wt_hbm, re_hbm, ce_hbm,
               ridx, cidx, h_buf, w_buf, sem_h, sem_w):
    wid = lax.axis_index("s") * NC + lax.axis_index("c")
    pltpu.sync_copy(rows_hbm, ridx)
    pltpu.sync_copy(cols_hbm, cidx)
    ga = pltpu.async_copy(ht_hbm.at[ridx.at[wid]], h_buf, sem_h)
    gb = pltpu.async_copy(wt_hbm.at[cidx.at[wid]], w_buf, sem_w)
    base = wid * RPW
    ga.wait()
    sa = pltpu.async_copy(h_buf, re_hbm.at[pl.ds(base, RPW), :], sem_h)
    gb.wait()
    sb = pltpu.async_copy(w_buf, ce_hbm.at[pl.ds(base, RPW), :], sem_w)
    sa.wait()
    sb.wait()


def _add_body(re_ref, ce_ref, o_ref):
    r = re_ref[...]
    c = ce_ref[...]
    o_ref[...] = (r[:, None, :] + c[None, :, :]).reshape(BH * W, D)


_add_tc = pl.pallas_call(
    _add_body,
    grid=(H // BH,),
    in_specs=[
        pl.BlockSpec((BH, D), lambda i: (i, 0)),
        pl.BlockSpec((W, D), lambda i: (0, 0)),
    ],
    out_specs=pl.BlockSpec((BH * W, D), lambda i: (i, 0)),
    out_shape=jax.ShapeDtypeStruct((H * W, D), jnp.float32),
    compiler_params=pltpu.CompilerParams(
        dimension_semantics=("arbitrary",)),
)


def kernel(height_table, width_table, shape):
    h = height_table.shape[0]
    w = width_table.shape[0]
    rows = jnp.minimum(jnp.arange(h, dtype=jnp.int32), shape[0] - 1)
    cols = jnp.minimum(jnp.arange(w, dtype=jnp.int32), shape[1] - 1)
    row_embed, col_embed = _lookup_sc(
        rows.astype(jnp.int32).reshape(NW, RPW),
        cols.astype(jnp.int32).reshape(NW, RPW),
        height_table, width_table)
    return _add_tc(row_embed, col_embed)


# speedup vs baseline: 1.3757x; 1.0088x over previous
import functools

import jax
import jax.numpy as jnp
from jax import lax
from jax.experimental import pallas as pl
from jax.experimental.pallas import tpu as pltpu
from jax.experimental.pallas import tpu_sc as plsc

H, W, D = 256, 256, 256
NC, NS, L = 2, 16, 16
NW = NC * NS
RPW = H // NW
BH = 8

_mesh = plsc.VectorSubcoreMesh(core_axis_name="c", subcore_axis_name="s")


@functools.partial(
    pl.kernel,
    out_type=(jax.ShapeDtypeStruct((H, D), jnp.float32),
              jax.ShapeDtypeStruct((W, D), jnp.float32)),
    mesh=_mesh,
    scratch_types=[
        pltpu.VMEM((NW, RPW), jnp.int32),
        pltpu.VMEM((NW, RPW), jnp.int32),
        pltpu.VMEM((RPW, D), jnp.float32),
        pltpu.VMEM((RPW, D), jnp.float32),
        pltpu.SemaphoreType.DMA,
        pltpu.SemaphoreType.DMA,
    ],
)
def _lookup_sc(rows_hbm, cols_hbm, ht_hbm, wt_hbm, re_hbm, ce_hbm,
               ridx, cidx, h_buf, w_buf, sem_h, sem_w):
    wid = lax.axis_index("s") * NC + lax.axis_index("c")
    pltpu.sync_copy(rows_hbm, ridx)
    pltpu.sync_copy(cols_hbm, cidx)
    ga = pltpu.async_copy(ht_hbm.at[ridx.at[wid]], h_buf, sem_h)
    gb = pltpu.async_copy(wt_hbm.at[cidx.at[wid]], w_buf, sem_w)
    base = wid * RPW
    ga.wait()
    sa = pltpu.async_copy(h_buf, re_hbm.at[pl.ds(base, RPW), :], sem_h)
    gb.wait()
    sb = pltpu.async_copy(w_buf, ce_hbm.at[pl.ds(base, RPW), :], sem_w)
    sa.wait()
    sb.wait()


def _add_body(re_ref, ce_ref, o_ref):
    c = ce_ref[...]
    for b in range(BH):
        o_ref[pl.ds(b * W, W), :] = c + re_ref[b, :][None, :]


_add_tc = pl.pallas_call(
    _add_body,
    grid=(H // BH,),
    in_specs=[
        pl.BlockSpec((BH, D), lambda i: (i, 0)),
        pl.BlockSpec((W, D), lambda i: (0, 0)),
    ],
    out_specs=pl.BlockSpec((BH * W, D), lambda i: (i, 0)),
    out_shape=jax.ShapeDtypeStruct((H * W, D), jnp.float32),
    compiler_params=pltpu.CompilerParams(
        dimension_semantics=("arbitrary",)),
)


def kernel(height_table, width_table, shape):
    h = height_table.shape[0]
    w = width_table.shape[0]
    rows = jnp.minimum(jnp.arange(h, dtype=jnp.int32), shape[0] - 1)
    cols = jnp.minimum(jnp.arange(w, dtype=jnp.int32), shape[1] - 1)
    row_embed, col_embed = _lookup_sc(
        rows.astype(jnp.int32).reshape(NW, RPW),
        cols.astype(jnp.int32).reshape(NW, RPW),
        height_table, width_table)
    return _add_tc(row_embed, col_embed)


# speedup vs baseline: 1.5781x; 1.1471x over previous
import functools

import jax
import jax.numpy as jnp
from jax import lax
from jax.experimental import pallas as pl
from jax.experimental.pallas import tpu as pltpu
from jax.experimental.pallas import tpu_sc as plsc

H, W, D = 256, 256, 256
NC, NS, L = 2, 16, 16
NW = NC * NS
RPW = H // NW
BH = 16

_mesh = plsc.VectorSubcoreMesh(core_axis_name="c", subcore_axis_name="s")


@functools.partial(
    pl.kernel,
    out_type=(jax.ShapeDtypeStruct((H, D), jnp.float32),
              jax.ShapeDtypeStruct((W, D), jnp.float32)),
    mesh=_mesh,
    scratch_types=[
        pltpu.VMEM((NW, RPW), jnp.int32),
        pltpu.VMEM((NW, RPW), jnp.int32),
        pltpu.VMEM((RPW, D), jnp.float32),
        pltpu.VMEM((RPW, D), jnp.float32),
        pltpu.SemaphoreType.DMA,
        pltpu.SemaphoreType.DMA,
    ],
)
def _lookup_sc(rows_hbm, cols_hbm, ht_hbm, wt_hbm, re_hbm, ce_hbm,
               ridx, cidx, h_buf, w_buf, sem_h, sem_w):
    wid = lax.axis_index("s") * NC + lax.axis_index("c")
    pltpu.sync_copy(rows_hbm, ridx)
    pltpu.sync_copy(cols_hbm, cidx)
    ga = pltpu.async_copy(ht_hbm.at[ridx.at[wid]], h_buf, sem_h)
    gb = pltpu.async_copy(wt_hbm.at[cidx.at[wid]], w_buf, sem_w)
    base = wid * RPW
    ga.wait()
    sa = pltpu.async_copy(h_buf, re_hbm.at[pl.ds(base, RPW), :], sem_h)
    gb.wait()
    sb = pltpu.async_copy(w_buf, ce_hbm.at[pl.ds(base, RPW), :], sem_w)
    sa.wait()
    sb.wait()


def _add_body(re_ref, ce_ref, o_ref):
    c = ce_ref[...]
    for b in range(BH):
        o_ref[pl.ds(b * W, W), :] = c + re_ref[b, :][None, :]


_add_tc = pl.pallas_call(
    _add_body,
    grid=(H // BH,),
    in_specs=[
        pl.BlockSpec((BH, D), lambda i: (i, 0)),
        pl.BlockSpec((W, D), lambda i: (0, 0)),
    ],
    out_specs=pl.BlockSpec((BH * W, D), lambda i: (i, 0)),
    out_shape=jax.ShapeDtypeStruct((H * W, D), jnp.float32),
    compiler_params=pltpu.CompilerParams(
        dimension_semantics=("arbitrary",)),
)


def kernel(height_table, width_table, shape):
    h = height_table.shape[0]
    w = width_table.shape[0]
    rows = jnp.minimum(jnp.arange(h, dtype=jnp.int32), shape[0] - 1)
    cols = jnp.minimum(jnp.arange(w, dtype=jnp.int32), shape[1] - 1)
    row_embed, col_embed = _lookup_sc(
        rows.astype(jnp.int32).reshape(NW, RPW),
        cols.astype(jnp.int32).reshape(NW, RPW),
        height_table, width_table)
    return _add_tc(row_embed, col_embed)
